# baseline (speedup 1.0000x reference)
import functools

import jax
import jax.numpy as jnp
from jax import lax
from jax.experimental import pallas as pl
from jax.experimental.pallas import tpu as pltpu
from jax.experimental.pallas import tpu_sc as plsc

N_CLASSES = 100000
EMBED_DIM = 64
BATCH = 16384

_NUM_CORES = 2
_NUM_SUBCORES = 16
_NW = _NUM_CORES * _NUM_SUBCORES
_B_PER_W = BATCH // _NW
_K = 16
_W_ELEMS = _B_PER_W * EMBED_DIM

_mesh = plsc.VectorSubcoreMesh(core_axis_name="c", subcore_axis_name="s")


@functools.partial(
    pl.kernel,
    mesh=_mesh,
    out_type=jax.ShapeDtypeStruct((BATCH * EMBED_DIM,), jnp.float32),
    scratch_types=[
        pltpu.VMEM((_B_PER_W,), jnp.int32),
        pltpu.VMEM((_W_ELEMS,), jnp.float32),
        pltpu.SemaphoreType.DMA,
    ],
)
def _embed_lookup(y_hbm, table_hbm, out_hbm, idx_v, rows_v, sem):
    wid = lax.axis_index("s") * _NUM_CORES + lax.axis_index("c")
    base = wid * _B_PER_W
    pltpu.sync_copy(y_hbm.at[pl.ds(base, _B_PER_W)], idx_v)

    def fire(c, _):
        j0 = c * _K
        vec = idx_v[pl.ds(j0, _K)]
        for j in range(_K):
            off = pl.multiple_of(vec[j] * EMBED_DIM, EMBED_DIM)
            pltpu.async_copy(
                table_hbm.at[pl.ds(off, EMBED_DIM)],
                rows_v.at[pl.ds((j0 + j) * EMBED_DIM, EMBED_DIM)],
                sem,
            )
        return ()

    lax.fori_loop(0, _B_PER_W // _K, fire, (), unroll=False)
    pltpu.make_async_copy(table_hbm.at[pl.ds(0, _W_ELEMS)], rows_v, sem).wait()
    pltpu.sync_copy(rows_v, out_hbm.at[pl.ds(base * EMBED_DIM, _W_ELEMS)])


def kernel(y, table):
    out = _embed_lookup(y.astype(jnp.int32), table.reshape(-1))
    return out.reshape(BATCH, EMBED_DIM)


# speedup vs baseline: 1.9082x; 1.9082x over previous
import functools

import jax
import jax.numpy as jnp
from jax import lax
from jax.experimental import pallas as pl
from jax.experimental.pallas import tpu as pltpu
from jax.experimental.pallas import tpu_sc as plsc

N_CLASSES = 100000
EMBED_DIM = 64
BATCH = 16384
_V = N_CLASSES + 1

_NUM_CORES = 2
_NUM_SUBCORES = 16
_NW = _NUM_CORES * _NUM_SUBCORES
_N_PASS = EMBED_DIM // _NW
_CHUNK = 2048
_NCH = BATCH // _CHUNK

_mesh = plsc.VectorSubcoreMesh(core_axis_name="c", subcore_axis_name="s")


@functools.partial(
    pl.kernel,
    mesh=_mesh,
    out_type=jax.ShapeDtypeStruct((EMBED_DIM, BATCH), jnp.float32),
    scratch_types=[
        pltpu.VMEM((_V,), jnp.float32),
        pltpu.VMEM((BATCH,), jnp.int32),
        pltpu.VMEM((_CHUNK,), jnp.float32),
        pltpu.VMEM((_CHUNK,), jnp.float32),
        pltpu.SemaphoreType.DMA,
        pltpu.SemaphoreType.DMA,
    ],
    compiler_params=pltpu.CompilerParams(needs_layout_passes=False),
)
def _embed_lookup(y_hbm, tT_hbm, oT_hbm, row_v, idx_v, obuf0, obuf1, sem_r, sem_w):
    wid = lax.axis_index("s") * _NUM_CORES + lax.axis_index("c")
    pltpu.sync_copy(y_hbm, idx_v)

    for p in range(_N_PASS):
        j = wid + _NW * p
        pltpu.async_copy(tT_hbm.at[j], row_v, sem_r).wait()

        for k in range(_NCH):
            buf = obuf0 if k % 2 == 0 else obuf1
            if k >= 2:
                pltpu.make_async_copy(oT_hbm.at[j, pl.ds(0, _CHUNK)], buf, sem_w).wait()
            c0 = k * _CHUNK

            def g16(k2, _, c0=c0, buf=buf):
                b0 = k2 * 16
                iv = idx_v[pl.ds(c0 + b0, 16)]
                buf[pl.ds(b0, 16)] = plsc.load_gather(row_v, [iv])
                return ()

            lax.fori_loop(0, _CHUNK // 16, g16, (), unroll=4)
            pltpu.async_copy(buf, oT_hbm.at[j, pl.ds(c0, _CHUNK)], sem_w)

        for _ in range(2):
            pltpu.make_async_copy(
                oT_hbm.at[j, pl.ds(0, _CHUNK)], obuf0, sem_w
            ).wait()


def kernel(y, table):
    out_t = _embed_lookup(y.astype(jnp.int32), table.T)
    return out_t.T


# speedup vs baseline: 2.6974x; 1.4136x over previous
import functools

import jax
import jax.numpy as jnp
from jax import lax
from jax.experimental import pallas as pl
from jax.experimental.pallas import tpu as pltpu
from jax.experimental.pallas import tpu_sc as plsc

N_CLASSES = 100000
EMBED_DIM = 64
BATCH = 16384
_V = N_CLASSES + 1

_NUM_CORES = 2
_NUM_SUBCORES = 16
_NW = _NUM_CORES * _NUM_SUBCORES
_N_PASS = EMBED_DIM // _NW
_CHUNK = 2048
_NCH = BATCH // _CHUNK

_mesh = plsc.VectorSubcoreMesh(core_axis_name="c", subcore_axis_name="s")


@functools.partial(
    pl.kernel,
    mesh=_mesh,
    out_type=jax.ShapeDtypeStruct((EMBED_DIM, BATCH), jnp.float32),
    scratch_types=[
        pltpu.VMEM((_V,), jnp.float32),
        pltpu.VMEM((BATCH,), jnp.int32),
        pltpu.VMEM((_CHUNK,), jnp.float32),
        pltpu.VMEM((_CHUNK,), jnp.float32),
        pltpu.SemaphoreType.DMA,
        pltpu.SemaphoreType.DMA,
    ],
    compiler_params=pltpu.CompilerParams(
        needs_layout_passes=False,
        disable_bounds_checks=True,
        disable_semaphore_checks=True,
    ),
)
def _embed_lookup(y_hbm, tT_hbm, oT_hbm, row_v, idx_v, obuf0, obuf1, sem_r, sem_w):
    wid = lax.axis_index("s") * _NUM_CORES + lax.axis_index("c")
    ycopy = pltpu.async_copy(y_hbm, idx_v, sem_r)

    for p in range(_N_PASS):
        j = wid + _NW * p
        rcopy = pltpu.async_copy(tT_hbm.at[j], row_v, sem_r)
        if p == 0:
            ycopy.wait()
        rcopy.wait()

        for k in range(_NCH):
            buf = obuf0 if k % 2 == 0 else obuf1
            if k >= 2:
                pltpu.make_async_copy(oT_hbm.at[j, pl.ds(0, _CHUNK)], buf, sem_w).wait()
            c0 = k * _CHUNK

            @plsc.parallel_loop(0, _CHUNK, step=16, unroll=4)
            def g16(b0, c0=c0, buf=buf):
                iv = idx_v[pl.ds(c0 + b0, 16)]
                buf[pl.ds(b0, 16)] = plsc.load_gather(row_v, [iv])
            pltpu.async_copy(buf, oT_hbm.at[j, pl.ds(c0, _CHUNK)], sem_w)

        for _ in range(2):
            pltpu.make_async_copy(
                oT_hbm.at[j, pl.ds(0, _CHUNK)], obuf0, sem_w
            ).wait()


def kernel(y, table):
    out_t = _embed_lookup(y.astype(jnp.int32), table.T)
    return out_t.T


# speedup vs baseline: 2.7665x; 1.0256x over previous
import functools

import jax
import jax.numpy as jnp
from jax import lax
from jax.experimental import pallas as pl
from jax.experimental.pallas import tpu as pltpu
from jax.experimental.pallas import tpu_sc as plsc

N_CLASSES = 100000
EMBED_DIM = 64
BATCH = 16384
_V = N_CLASSES + 1

_NUM_CORES = 2
_NUM_SUBCORES = 16
_NW = _NUM_CORES * _NUM_SUBCORES
_N_PASS = EMBED_DIM // _NW
_CHUNK = 4096
_NCH = BATCH // _CHUNK

_mesh = plsc.VectorSubcoreMesh(core_axis_name="c", subcore_axis_name="s")


@functools.partial(
    pl.kernel,
    mesh=_mesh,
    out_type=jax.ShapeDtypeStruct((EMBED_DIM, BATCH), jnp.float32),
    scratch_types=[
        pltpu.VMEM((_V,), jnp.float32),
        pltpu.VMEM((BATCH,), jnp.int32),
        pltpu.VMEM((_CHUNK,), jnp.float32),
        pltpu.VMEM((_CHUNK,), jnp.float32),
        pltpu.SemaphoreType.DMA,
        pltpu.SemaphoreType.DMA,
    ],
    compiler_params=pltpu.CompilerParams(
        needs_layout_passes=False,
        disable_bounds_checks=True,
        disable_semaphore_checks=True,
        skip_device_barrier=True,
    ),
)
def _embed_lookup(y_hbm, tT_hbm, oT_hbm, row_v, idx_v, obuf0, obuf1, sem_r, sem_w):
    wid = lax.axis_index("s") * _NUM_CORES + lax.axis_index("c")
    ycopy = pltpu.async_copy(y_hbm, idx_v, sem_r)

    for p in range(_N_PASS):
        j = wid + _NW * p
        rcopy = pltpu.async_copy(tT_hbm.at[j], row_v, sem_r)
        if p == 0:
            ycopy.wait()
        rcopy.wait()

        for k in range(_NCH):
            buf = obuf0 if k % 2 == 0 else obuf1
            if k >= 2:
                pltpu.make_async_copy(oT_hbm.at[j, pl.ds(0, _CHUNK)], buf, sem_w).wait()
            c0 = k * _CHUNK

            @plsc.parallel_loop(0, _CHUNK, step=16, unroll=8)
            def g16(b0, c0=c0, buf=buf):
                iv = idx_v[pl.ds(c0 + b0, 16)]
                buf[pl.ds(b0, 16)] = plsc.load_gather(row_v, [iv])
            pltpu.async_copy(buf, oT_hbm.at[j, pl.ds(c0, _CHUNK)], sem_w)

        for _ in range(2):
            pltpu.make_async_copy(
                oT_hbm.at[j, pl.ds(0, _CHUNK)], obuf0, sem_w
            ).wait()


def kernel(y, table):
    out_t = _embed_lookup(y.astype(jnp.int32), table.T)
    return out_t.T


# speedup vs baseline: 2.7854x; 1.0069x over previous
import functools

import jax
import jax.numpy as jnp
from jax import lax
from jax.experimental import pallas as pl
from jax.experimental.pallas import tpu as pltpu
from jax.experimental.pallas import tpu_sc as plsc

N_CLASSES = 100000
EMBED_DIM = 64
BATCH = 16384
_V = N_CLASSES + 1

_NUM_CORES = 2
_NUM_SUBCORES = 16
_NW = _NUM_CORES * _NUM_SUBCORES
_N_PASS = EMBED_DIM // _NW
_CHUNK = 4096
_NCH = BATCH // _CHUNK

_mesh = plsc.VectorSubcoreMesh(core_axis_name="c", subcore_axis_name="s")


@functools.partial(
    pl.kernel,
    mesh=_mesh,
    out_type=jax.ShapeDtypeStruct((EMBED_DIM, BATCH), jnp.float32),
    scratch_types=[
        pltpu.VMEM((_V,), jnp.float32),
        pltpu.VMEM((BATCH,), jnp.int32),
        pltpu.VMEM((_CHUNK,), jnp.float32),
        pltpu.VMEM((_CHUNK,), jnp.float32),
        pltpu.SemaphoreType.DMA,
        pltpu.SemaphoreType.DMA,
    ],
    compiler_params=pltpu.CompilerParams(
        needs_layout_passes=False,
        disable_bounds_checks=True,
        disable_semaphore_checks=True,
        skip_device_barrier=True,
    ),
)
def _embed_lookup(y_hbm, tT_hbm, oT_hbm, row_v, idx_v, obuf0, obuf1, sem_r, sem_w):
    wid = lax.axis_index("s") * _NUM_CORES + lax.axis_index("c")
    ycopy = pltpu.async_copy(y_hbm, idx_v, sem_r)

    rcopy = pltpu.async_copy(tT_hbm.at[wid], row_v, sem_r)
    ycopy.wait()
    for p in range(_N_PASS):
        j = wid + _NW * p
        rcopy.wait()

        for k in range(_NCH):
            buf = obuf0 if k % 2 == 0 else obuf1
            if k >= 2:
                pltpu.make_async_copy(oT_hbm.at[j, pl.ds(0, _CHUNK)], buf, sem_w).wait()
            c0 = k * _CHUNK

            @plsc.parallel_loop(0, _CHUNK, step=16, unroll=8)
            def g16(b0, c0=c0, buf=buf):
                iv = idx_v[pl.ds(c0 + b0, 16)]
                buf[pl.ds(b0, 16)] = plsc.load_gather(row_v, [iv])
            pltpu.async_copy(buf, oT_hbm.at[j, pl.ds(c0, _CHUNK)], sem_w)

        if p + 1 < _N_PASS:
            rcopy = pltpu.async_copy(tT_hbm.at[wid + _NW * (p + 1)], row_v, sem_r)

        for _ in range(2):
            pltpu.make_async_copy(
                oT_hbm.at[j, pl.ds(0, _CHUNK)], obuf0, sem_w
            ).wait()


def kernel(y, table):
    out_t = _embed_lookup(y.astype(jnp.int32), table.T)
    return out_t.T
